# baseline (speedup 1.0000x reference)
import functools

import jax
import jax.numpy as jnp
from jax import lax
from jax.experimental import pallas as pl
from jax.experimental.pallas import tpu as pltpu
from jax.experimental.pallas import tpu_sc as plsc

N = 10000
E = 320000
D = 128
DH = D // 2
NC = 2
NS = 16
K = 80

NW = NC * NS
EPW = E // NW
NBLK_D = EPW // K

EPS = E // NS
NBLK_S = EPS // K

RPT = N // NS

_MESH = plsc.VectorSubcoreMesh(core_axis_name="c", subcore_axis_name="s")
_SC_PARAMS = pltpu.CompilerParams(needs_layout_passes=False,
                                  use_tc_tiling_on_sc=False)


@functools.partial(
    pl.kernel,
    out_type=jax.ShapeDtypeStruct((NC, 1, N), jnp.float32),
    mesh=_MESH,
    scratch_types=[
        pltpu.VMEM((NBLK_D, K), jnp.int32),
        pltpu.VMEM((NBLK_D, K), jnp.float32),
        pltpu.VMEM((2000,), jnp.float32),
        pltpu.VMEM_SHARED((N,), jnp.float32),
    ],
    compiler_params=_SC_PARAMS,
)
def _sc_degree(dst_hbm, ew_hbm, out_hbm, dstv, ewv, zbuf, acc):
    cid = lax.axis_index("c")
    sid = lax.axis_index("s")
    wid = cid * NS + sid

    zero16 = jnp.zeros((16,), jnp.float32)

    def _z(i, _):
        zbuf[pl.ds(i * 16, 16)] = zero16
        return 0
    lax.fori_loop(0, 125, _z, 0)

    @pl.when(sid == 0)
    def _():
        for k in range(5):
            pltpu.sync_copy(zbuf, acc.at[pl.ds(k * 2000, 2000)])

    pltpu.sync_copy(dst_hbm.at[wid], dstv)
    pltpu.sync_copy(ew_hbm.at[wid], ewv)
    plsc.subcore_barrier()

    def _blk(j, _):
        pltpu.sync_copy(ewv.at[j], acc.at[dstv.at[j]], add=True)
        return 0
    lax.fori_loop(0, NBLK_D, _blk, 0)

    plsc.subcore_barrier()

    @pl.when(sid == 0)
    def _():
        pltpu.sync_copy(acc, out_hbm.at[cid, 0])


@functools.partial(
    pl.kernel,
    out_type=jax.ShapeDtypeStruct((NC, N, DH), jnp.float32),
    mesh=_MESH,
    scratch_types=[
        pltpu.VMEM((EPS,), jnp.int32),
        pltpu.VMEM((NBLK_S, K), jnp.int32),
        pltpu.VMEM((EPS,), jnp.float32),
        [pltpu.VMEM((K, DH), jnp.bfloat16) for _ in range(5)],
        [pltpu.VMEM((K, DH), jnp.float32) for _ in range(2)],
        pltpu.VMEM_SHARED((N, DH), jnp.float32),
        [pltpu.SemaphoreType.DMA for _ in range(5)],
        [pltpu.SemaphoreType.DMA for _ in range(2)],
    ],
    compiler_params=_SC_PARAMS,
)
def _sc_scatter(y_hbm, src_hbm, dst_hbm, ew_hbm, out_hbm,
                srcv, dstv, ewv, gbufs, fbufs, acc, gsems, ssems):
    cid = lax.axis_index("c")
    sid = lax.axis_index("s")

    pltpu.sync_copy(src_hbm.at[pl.ds(sid * EPS, EPS)], srcv)
    pltpu.sync_copy(dst_hbm.at[sid], dstv)
    pltpu.sync_copy(ew_hbm.at[pl.ds(sid * EPS, EPS)], ewv)

    zero16 = jnp.zeros((16,), jnp.float32)

    def _zrow(r, _):
        for cc in range(DH // 16):
            fbufs[0][r, pl.ds(cc * 16, 16)] = zero16
        return 0
    lax.fori_loop(0, K, _zrow, 0)

    base = sid * RPT
    for k in range(7):
        pltpu.sync_copy(fbufs[0], acc.at[pl.ds(base + k * K, K)])
    pltpu.sync_copy(fbufs[0].at[pl.ds(0, RPT - 7 * K)],
                    acc.at[pl.ds(base + 7 * K, RPT - 7 * K)])
    plsc.subcore_barrier()

    yv = y_hbm.at[cid]

    def _gather(j, buf, sem):
        off = pl.multiple_of(j * K, 8)
        pltpu.async_copy(yv.at[srcv.at[pl.ds(off, K)]], buf, sem)

    def _wait_gather(buf, sem):
        pltpu.make_async_copy(yv.at[srcv.at[pl.ds(0, K)]], buf, sem).wait()

    def _scatter(j, buf, sem):
        pltpu.async_copy(buf, acc.at[dstv.at[j]], sem, add=True)

    def _wait_scatter(buf, sem):
        pltpu.make_async_copy(buf, acc.at[dstv.at[0]], sem).wait()

    iota2 = lax.iota(jnp.int32, 16) * 2

    def _scale(j, gbuf, fbuf):
        @plsc.parallel_loop(0, K, step=16, unroll=1)
        def _rows16(r0):
            c16 = ewv[pl.ds(j * K + r0, 16)]
            for lane in range(16):
                crep = jnp.full((16,), c16[lane])
                r16 = jnp.full((16,), r0 + lane, jnp.int32)
                for c2 in range(DH // 32):
                    ab = gbuf[r0 + lane, pl.ds(c2 * 32, 32)]
                    a, b = plsc.unpack(ab, format=plsc.PackFormat.INTERLEAVED)
                    plsc.store_scatter(fbuf, [r16, c2 * 32 + iota2],
                                       a * crep)
                    plsc.store_scatter(fbuf, [r16, c2 * 32 + 1 + iota2],
                                       b * crep)

    for b in range(4):
        _gather(b, gbufs[b], gsems[b])

    def _decade(jq, _):
        for d in range(10):
            j = jq * 10 + d
            g = d % 5
            f = d % 2
            gp = (d + 4) % 5
            _wait_gather(gbufs[g], gsems[g])
            if d < 2:
                @pl.when(jq > 0)
                def _():
                    _wait_scatter(fbufs[f], ssems[f])
            else:
                _wait_scatter(fbufs[f], ssems[f])
            _scale(j, gbufs[g], fbufs[f])
            _scatter(j, fbufs[f], ssems[f])
            if d < 6:
                _gather(j + 4, gbufs[gp], gsems[gp])
            else:
                @pl.when(jq < NBLK_S // 10 - 1)
                def _():
                    _gather(j + 4, gbufs[gp], gsems[gp])
        return 0
    lax.fori_loop(0, NBLK_S // 10, _decade, 0)

    for b in range(2):
        _wait_scatter(fbufs[b], ssems[b])

    plsc.subcore_barrier()

    @pl.when(sid < 10)
    def _():
        pltpu.sync_copy(acc.at[pl.ds(sid * 1000, 1000)],
                        out_hbm.at[cid, pl.ds(sid * 1000, 1000)])


def _tc_prep_body(degp_ref, x_ref, w_ref, y_ref, dis_ref):
    d2 = degp_ref[...]
    deg = d2[:, 0:1] + d2[:, 1:2] + 1.0
    dis = lax.rsqrt(deg)
    dis_ref[...] = dis
    yw = jnp.dot(x_ref[...], w_ref[...],
                 preferred_element_type=jnp.float32) * dis
    ybf = yw.astype(jnp.bfloat16)
    y_ref[0] = ybf[:, :DH]
    y_ref[1] = ybf[:, DH:]


_tc_prep = pl.pallas_call(
    _tc_prep_body,
    out_shape=[jax.ShapeDtypeStruct((NC, N, DH), jnp.bfloat16),
               jax.ShapeDtypeStruct((N, 1), jnp.float32)],
)


def _tc_mid_body(s_ref, y_ref, dis_ref, b_ref, w_ref, y2_ref):
    s = jnp.concatenate([s_ref[0], s_ref[1]], axis=1)
    y = jnp.concatenate([y_ref[0], y_ref[1]], axis=1).astype(jnp.float32)
    dis = dis_ref[...]
    h = jnp.maximum(dis * (s + y) + b_ref[...], 0.0)
    y2 = jnp.dot(h, w_ref[...], preferred_element_type=jnp.float32) * dis
    y2bf = y2.astype(jnp.bfloat16)
    y2_ref[0] = y2bf[:, :DH]
    y2_ref[1] = y2bf[:, DH:]


_tc_mid = pl.pallas_call(
    _tc_mid_body,
    out_shape=jax.ShapeDtypeStruct((NC, N, DH), jnp.bfloat16),
)


def _tc_final_body(s_ref, y2_ref, dis_ref, b_ref, out_ref):
    s = jnp.concatenate([s_ref[0], s_ref[1]], axis=1)
    y2 = jnp.concatenate([y2_ref[0], y2_ref[1]], axis=1).astype(jnp.float32)
    z = dis_ref[...] * (s + y2) + b_ref[...]
    m = jnp.max(z, axis=1, keepdims=True)
    lse = jnp.log(jnp.sum(jnp.exp(z - m), axis=1, keepdims=True)) + m
    out_ref[...] = z - lse


_tc_final = pl.pallas_call(
    _tc_final_body,
    out_shape=jax.ShapeDtypeStruct((N, D), jnp.float32),
)


def kernel(x, edge_index, edge_attr, W1, b1, W2, b2):
    dst_d = edge_index[1].reshape(NW, NBLK_D, K)
    ew_d = edge_attr.reshape(NW, NBLK_D, K)
    src_s = edge_index[0]
    dst_s = edge_index[1].reshape(NS, NBLK_S, K)

    degp = _sc_degree(dst_d, ew_d)
    y1, dis = _tc_prep(degp[:, 0, :].T, x, W1)
    s1 = _sc_scatter(y1, src_s, dst_s, edge_attr)
    y2 = _tc_mid(s1, y1, dis, b1.reshape(1, D), W2)
    s2 = _sc_scatter(y2, src_s, dst_s, edge_attr)
    return _tc_final(s2, y2, dis, b2.reshape(1, D))


# speedup vs baseline: 1.2085x; 1.2085x over previous
import functools

import jax
import jax.numpy as jnp
from jax import lax
from jax.experimental import pallas as pl
from jax.experimental.pallas import tpu as pltpu
from jax.experimental.pallas import tpu_sc as plsc

N = 10000
E = 320000
D = 128
DH = D // 2
NC = 2
NS = 16
K = 80

NW = NC * NS
EPW = E // NW
NBLK_D = EPW // K

EPS = E // NS
NBLK_S = EPS // K

RPT = N // NS

_MESH = plsc.VectorSubcoreMesh(core_axis_name="c", subcore_axis_name="s")
_SC_PARAMS = pltpu.CompilerParams(needs_layout_passes=False,
                                  use_tc_tiling_on_sc=False)


@functools.partial(
    pl.kernel,
    out_type=jax.ShapeDtypeStruct((NC, 1, N), jnp.float32),
    mesh=_MESH,
    scratch_types=[
        pltpu.VMEM((NBLK_D, K), jnp.int32),
        pltpu.VMEM((NBLK_D, K), jnp.float32),
        pltpu.VMEM((2000,), jnp.float32),
        pltpu.VMEM_SHARED((N,), jnp.float32),
    ],
    compiler_params=_SC_PARAMS,
)
def _sc_degree(dst_hbm, ew_hbm, out_hbm, dstv, ewv, zbuf, acc):
    cid = lax.axis_index("c")
    sid = lax.axis_index("s")
    wid = cid * NS + sid

    zero16 = jnp.zeros((16,), jnp.float32)

    def _z(i, _):
        zbuf[pl.ds(i * 16, 16)] = zero16
        return 0
    lax.fori_loop(0, 125, _z, 0)

    @pl.when(sid == 0)
    def _():
        for k in range(5):
            pltpu.sync_copy(zbuf, acc.at[pl.ds(k * 2000, 2000)])

    pltpu.sync_copy(dst_hbm.at[wid], dstv)
    pltpu.sync_copy(ew_hbm.at[wid], ewv)
    plsc.subcore_barrier()

    def _blk(j, _):
        pltpu.sync_copy(ewv.at[j], acc.at[dstv.at[j]], add=True)
        return 0
    lax.fori_loop(0, NBLK_D, _blk, 0)

    plsc.subcore_barrier()

    @pl.when(sid == 0)
    def _():
        pltpu.sync_copy(acc, out_hbm.at[cid, 0])


@functools.partial(
    pl.kernel,
    out_type=jax.ShapeDtypeStruct((NC, N, DH), jnp.float32),
    mesh=_MESH,
    scratch_types=[
        pltpu.VMEM((EPS,), jnp.int32),
        pltpu.VMEM((NBLK_S, K), jnp.int32),
        pltpu.VMEM((EPS,), jnp.float32),
        [pltpu.VMEM((K, DH), jnp.bfloat16) for _ in range(4)],
        [pltpu.VMEM((K, DH), jnp.float32) for _ in range(4)],
        pltpu.VMEM_SHARED((N, DH), jnp.float32),
        [pltpu.SemaphoreType.DMA for _ in range(4)],
        [pltpu.SemaphoreType.DMA for _ in range(4)],
    ],
    compiler_params=_SC_PARAMS,
)
def _sc_scatter(y_hbm, src_hbm, dst_hbm, ew_hbm, out_hbm,
                srcv, dstv, ewv, gbufs, fbufs, acc, gsems, ssems):
    cid = lax.axis_index("c")
    sid = lax.axis_index("s")

    pltpu.sync_copy(src_hbm.at[pl.ds(sid * EPS, EPS)], srcv)
    pltpu.sync_copy(dst_hbm.at[sid], dstv)
    pltpu.sync_copy(ew_hbm.at[pl.ds(sid * EPS, EPS)], ewv)

    zero16 = jnp.zeros((16,), jnp.float32)

    def _zrow(r, _):
        for cc in range(DH // 16):
            fbufs[0][r, pl.ds(cc * 16, 16)] = zero16
        return 0
    lax.fori_loop(0, K, _zrow, 0)

    base = sid * RPT
    for k in range(7):
        pltpu.sync_copy(fbufs[0], acc.at[pl.ds(base + k * K, K)])
    pltpu.sync_copy(fbufs[0].at[pl.ds(0, RPT - 7 * K)],
                    acc.at[pl.ds(base + 7 * K, RPT - 7 * K)])
    plsc.subcore_barrier()

    yv = y_hbm.at[cid]

    def _gather(j, buf, sem):
        off = pl.multiple_of(j * K, 8)
        pltpu.async_copy(yv.at[srcv.at[pl.ds(off, K)]], buf, sem)

    def _wait_gather(buf, sem):
        pltpu.make_async_copy(yv.at[srcv.at[pl.ds(0, K)]], buf, sem).wait()

    def _scatter(j, buf, sem):
        pltpu.async_copy(buf, acc.at[dstv.at[j]], sem, add=True)

    def _wait_scatter(buf, sem):
        pltpu.make_async_copy(buf, acc.at[dstv.at[0]], sem).wait()

    iota2 = lax.iota(jnp.int32, 16) * 2

    def _scale(j, gbuf, fbuf):
        @plsc.parallel_loop(0, K, step=16, unroll=1)
        def _rows16(r0):
            c16 = ewv[pl.ds(j * K + r0, 16)]
            for lane in range(16):
                crep = jnp.full((16,), c16[lane])
                r16 = jnp.full((16,), r0 + lane, jnp.int32)
                for c2 in range(DH // 32):
                    ab = gbuf[r0 + lane, pl.ds(c2 * 32, 32)]
                    a, b = plsc.unpack(ab, format=plsc.PackFormat.INTERLEAVED)
                    plsc.store_scatter(fbuf, [r16, c2 * 32 + iota2],
                                       a * crep)
                    plsc.store_scatter(fbuf, [r16, c2 * 32 + 1 + iota2],
                                       b * crep)

    NQ = NBLK_S // 4
    for b in range(4):
        _gather(b, gbufs[b], gsems[b])

    def _quad(jq, _):
        for d in range(4):
            j = jq * 4 + d
            _wait_gather(gbufs[d], gsems[d])

            @pl.when(jq > 0)
            def _():
                _wait_scatter(fbufs[d], ssems[d])
            _scale(j, gbufs[d], fbufs[d])
            _scatter(j, fbufs[d], ssems[d])
            if d < 2:
                _gather(j + 4, gbufs[d], gsems[d])
            else:
                @pl.when(jq < NQ - 1)
                def _():
                    _gather(j + 4, gbufs[d], gsems[d])
        return 0
    lax.fori_loop(0, NQ, _quad, 0)

    for d in range(2):
        j = NQ * 4 + d
        _wait_gather(gbufs[d], gsems[d])
        _wait_scatter(fbufs[d], ssems[d])
        _scale(j, gbufs[d], fbufs[d])
        _scatter(j, fbufs[d], ssems[d])

    for b in [2, 3, 0, 1]:
        _wait_scatter(fbufs[b], ssems[b])

    plsc.subcore_barrier()

    @pl.when(sid < 10)
    def _():
        pltpu.sync_copy(acc.at[pl.ds(sid * 1000, 1000)],
                        out_hbm.at[cid, pl.ds(sid * 1000, 1000)])


def _tc_prep_body(degp_ref, x_ref, w_ref, y_ref, dis_ref):
    d2 = degp_ref[...]
    deg = d2[:, 0:1] + d2[:, 1:2] + 1.0
    dis = lax.rsqrt(deg)
    dis_ref[...] = dis
    yw = jnp.dot(x_ref[...], w_ref[...],
                 preferred_element_type=jnp.float32) * dis
    ybf = yw.astype(jnp.bfloat16)
    y_ref[0] = ybf[:, :DH]
    y_ref[1] = ybf[:, DH:]


_tc_prep = pl.pallas_call(
    _tc_prep_body,
    out_shape=[jax.ShapeDtypeStruct((NC, N, DH), jnp.bfloat16),
               jax.ShapeDtypeStruct((N, 1), jnp.float32)],
)


def _tc_mid_body(s_ref, y_ref, dis_ref, b_ref, w_ref, y2_ref):
    s = jnp.concatenate([s_ref[0], s_ref[1]], axis=1)
    y = jnp.concatenate([y_ref[0], y_ref[1]], axis=1).astype(jnp.float32)
    dis = dis_ref[...]
    h = jnp.maximum(dis * (s + y) + b_ref[...], 0.0)
    y2 = jnp.dot(h, w_ref[...], preferred_element_type=jnp.float32) * dis
    y2bf = y2.astype(jnp.bfloat16)
    y2_ref[0] = y2bf[:, :DH]
    y2_ref[1] = y2bf[:, DH:]


_tc_mid = pl.pallas_call(
    _tc_mid_body,
    out_shape=jax.ShapeDtypeStruct((NC, N, DH), jnp.bfloat16),
)


def _tc_final_body(s_ref, y2_ref, dis_ref, b_ref, out_ref):
    s = jnp.concatenate([s_ref[0], s_ref[1]], axis=1)
    y2 = jnp.concatenate([y2_ref[0], y2_ref[1]], axis=1).astype(jnp.float32)
    z = dis_ref[...] * (s + y2) + b_ref[...]
    m = jnp.max(z, axis=1, keepdims=True)
    lse = jnp.log(jnp.sum(jnp.exp(z - m), axis=1, keepdims=True)) + m
    out_ref[...] = z - lse


_tc_final = pl.pallas_call(
    _tc_final_body,
    out_shape=jax.ShapeDtypeStruct((N, D), jnp.float32),
)


def kernel(x, edge_index, edge_attr, W1, b1, W2, b2):
    dst_d = edge_index[1].reshape(NW, NBLK_D, K)
    ew_d = edge_attr.reshape(NW, NBLK_D, K)
    src_s = edge_index[0]
    dst_s = edge_index[1].reshape(NS, NBLK_S, K)

    degp = _sc_degree(dst_d, ew_d)
    y1, dis = _tc_prep(degp[:, 0, :].T, x, W1)
    s1 = _sc_scatter(y1, src_s, dst_s, edge_attr)
    y2 = _tc_mid(s1, y1, dis, b1.reshape(1, D), W2)
    s2 = _sc_scatter(y2, src_s, dst_s, edge_attr)
    return _tc_final(s2, y2, dis, b2.reshape(1, D))


# speedup vs baseline: 1.9165x; 1.5858x over previous
import functools

import jax
import jax.numpy as jnp
from jax import lax
from jax.experimental import pallas as pl
from jax.experimental.pallas import tpu as pltpu
from jax.experimental.pallas import tpu_sc as plsc

N = 10000
E = 320000
D = 128
DH = D // 2
NC = 2
NS = 16
K = 80

NW = NC * NS
EPW = E // NW
NBLK_D = EPW // K

EPS = E // NS
NBLK_S = EPS // K

RPT = N // NS

_MESH = plsc.VectorSubcoreMesh(core_axis_name="c", subcore_axis_name="s")
_SC_PARAMS = pltpu.CompilerParams(needs_layout_passes=False,
                                  use_tc_tiling_on_sc=False)


@functools.partial(
    pl.kernel,
    out_type=jax.ShapeDtypeStruct((NC, 1, N), jnp.float32),
    mesh=_MESH,
    scratch_types=[
        pltpu.VMEM((NBLK_D, K), jnp.int32),
        pltpu.VMEM((NBLK_D, K), jnp.float32),
        pltpu.VMEM((2000,), jnp.float32),
        pltpu.VMEM_SHARED((N,), jnp.float32),
        pltpu.SemaphoreType.DMA,
    ],
    compiler_params=_SC_PARAMS,
)
def _sc_degree(dst_hbm, ew_hbm, out_hbm, dstv, ewv, zbuf, acc, dsem):
    cid = lax.axis_index("c")
    sid = lax.axis_index("s")
    wid = cid * NS + sid

    zero16 = jnp.zeros((16,), jnp.float32)

    def _z(i, _):
        zbuf[pl.ds(i * 16, 16)] = zero16
        return 0
    lax.fori_loop(0, 125, _z, 0)

    @pl.when(sid == 0)
    def _():
        for k in range(5):
            pltpu.sync_copy(zbuf, acc.at[pl.ds(k * 2000, 2000)])

    pltpu.sync_copy(dst_hbm.at[wid], dstv)
    pltpu.sync_copy(ew_hbm.at[wid], ewv)
    plsc.subcore_barrier()

    def _blk5(g, _):
        for t in range(5):
            pltpu.async_copy(ewv.at[g * 5 + t], acc.at[dstv.at[g * 5 + t]],
                             dsem, add=True)
        for t in range(5):
            pltpu.make_async_copy(ewv.at[0], acc.at[dstv.at[0]],
                                  dsem).wait()
        return 0
    lax.fori_loop(0, NBLK_D // 5, _blk5, 0)

    plsc.subcore_barrier()

    @pl.when(sid == 0)
    def _():
        pltpu.sync_copy(acc, out_hbm.at[cid, 0])


@functools.partial(
    pl.kernel,
    out_type=jax.ShapeDtypeStruct((NC, N, DH), jnp.float32),
    mesh=_MESH,
    scratch_types=[
        pltpu.VMEM((NBLK_S, K), jnp.int32),
        pltpu.VMEM((NBLK_S, K), jnp.int32),
        pltpu.VMEM((EPS,), jnp.float32),
        [pltpu.VMEM((K, DH), jnp.float32) for _ in range(5)],
        pltpu.VMEM_SHARED((N, DH), jnp.float32),
        [pltpu.SemaphoreType.DMA for _ in range(5)],
        [pltpu.SemaphoreType.DMA for _ in range(5)],
    ],
    compiler_params=_SC_PARAMS,
)
def _sc_scatter(y_hbm, src_hbm, dst_hbm, ew_hbm, out_hbm,
                srcv, dstv, ewv, bufs, acc, gsems, ssems):
    cid = lax.axis_index("c")
    sid = lax.axis_index("s")

    pltpu.sync_copy(src_hbm.at[sid], srcv)
    pltpu.sync_copy(dst_hbm.at[sid], dstv)
    pltpu.sync_copy(ew_hbm.at[pl.ds(sid * EPS, EPS)], ewv)

    zero16 = jnp.zeros((16,), jnp.float32)

    def _zrow(r, _):
        for cc in range(DH // 16):
            bufs[0][r, pl.ds(cc * 16, 16)] = zero16
        return 0
    lax.fori_loop(0, K, _zrow, 0)

    base = sid * RPT
    for k in range(7):
        pltpu.sync_copy(bufs[0], acc.at[pl.ds(base + k * K, K)])
    pltpu.sync_copy(bufs[0].at[pl.ds(0, RPT - 7 * K)],
                    acc.at[pl.ds(base + 7 * K, RPT - 7 * K)])
    plsc.subcore_barrier()

    yv = y_hbm.at[cid]

    def _gather(j, buf, sem):
        pltpu.async_copy(yv.at[srcv.at[j]], buf, sem)

    def _wait_gather(buf, sem):
        pltpu.make_async_copy(yv.at[srcv.at[0]], buf, sem).wait()

    def _scatter(j, buf, sem):
        pltpu.async_copy(buf, acc.at[dstv.at[j]], sem, add=True)

    def _wait_scatter(buf, sem):
        pltpu.make_async_copy(buf, acc.at[dstv.at[0]], sem).wait()

    def _scale(j, buf):
        @plsc.parallel_loop(0, K, step=16, unroll=2)
        def _rows16(r0):
            c16 = ewv[pl.ds(j * K + r0, 16)]
            for lane in range(16):
                crep = jnp.full((16,), c16[lane])
                for cc in range(DH // 16):
                    buf[r0 + lane, pl.ds(cc * 16, 16)] = (
                        buf[r0 + lane, pl.ds(cc * 16, 16)] * crep)

    for b in range(4):
        _gather(b, bufs[b], gsems[b])

    def _quint(jq, _):
        for q in range(5):
            j = jq * 5 + q
            bp = (q + 4) % 5
            _wait_gather(bufs[q], gsems[q])
            _scale(j, bufs[q])
            _scatter(j, bufs[q], ssems[q])
            if q == 0:
                @pl.when(jq > 0)
                def _():
                    _wait_scatter(bufs[bp], ssems[bp])
                _gather(j + 4, bufs[bp], gsems[bp])
            else:
                _wait_scatter(bufs[bp], ssems[bp])

                @pl.when(jq < NBLK_S // 5 - 1)
                def _():
                    _gather(j + 4, bufs[bp], gsems[bp])
        return 0
    lax.fori_loop(0, NBLK_S // 5, _quint, 0)

    _wait_scatter(bufs[4], ssems[4])

    plsc.subcore_barrier()

    @pl.when(sid < 10)
    def _():
        pltpu.sync_copy(acc.at[pl.ds(sid * 1000, 1000)],
                        out_hbm.at[cid, pl.ds(sid * 1000, 1000)])


def _tc_prep_body(degp_ref, x_ref, w_ref, y_ref, dis_ref):
    d2 = degp_ref[...]
    deg = d2[:, 0:1] + d2[:, 1:2] + 1.0
    dis = lax.rsqrt(deg)
    dis_ref[...] = dis
    yw = jnp.dot(x_ref[...], w_ref[...],
                 preferred_element_type=jnp.float32) * dis
    y_ref[0] = yw[:, :DH]
    y_ref[1] = yw[:, DH:]


_tc_prep = pl.pallas_call(
    _tc_prep_body,
    out_shape=[jax.ShapeDtypeStruct((NC, N, DH), jnp.float32),
               jax.ShapeDtypeStruct((N, 1), jnp.float32)],
)


def _tc_mid_body(s_ref, y_ref, dis_ref, b_ref, w_ref, y2_ref):
    s = jnp.concatenate([s_ref[0], s_ref[1]], axis=1)
    y = jnp.concatenate([y_ref[0], y_ref[1]], axis=1)
    dis = dis_ref[...]
    h = jnp.maximum(dis * (s + y) + b_ref[...], 0.0)
    y2 = jnp.dot(h, w_ref[...], preferred_element_type=jnp.float32) * dis
    y2_ref[0] = y2[:, :DH]
    y2_ref[1] = y2[:, DH:]


_tc_mid = pl.pallas_call(
    _tc_mid_body,
    out_shape=jax.ShapeDtypeStruct((NC, N, DH), jnp.float32),
)


def _tc_final_body(s_ref, y2_ref, dis_ref, b_ref, out_ref):
    s = jnp.concatenate([s_ref[0], s_ref[1]], axis=1)
    y2 = jnp.concatenate([y2_ref[0], y2_ref[1]], axis=1)
    z = dis_ref[...] * (s + y2) + b_ref[...]
    m = jnp.max(z, axis=1, keepdims=True)
    lse = jnp.log(jnp.sum(jnp.exp(z - m), axis=1, keepdims=True)) + m
    out_ref[...] = z - lse


_tc_final = pl.pallas_call(
    _tc_final_body,
    out_shape=jax.ShapeDtypeStruct((N, D), jnp.float32),
)


def kernel(x, edge_index, edge_attr, W1, b1, W2, b2):
    dst_d = edge_index[1].reshape(NW, NBLK_D, K)
    ew_d = edge_attr.reshape(NW, NBLK_D, K)
    src_s = edge_index[0].reshape(NS, NBLK_S, K)
    dst_s = edge_index[1].reshape(NS, NBLK_S, K)

    degp = _sc_degree(dst_d, ew_d)
    y1, dis = _tc_prep(degp[:, 0, :].T, x, W1)
    s1 = _sc_scatter(y1, src_s, dst_s, edge_attr)
    y2 = _tc_mid(s1, y1, dis, b1.reshape(1, D), W2)
    s2 = _sc_scatter(y2, src_s, dst_s, edge_attr)
    return _tc_final(s2, y2, dis, b2.reshape(1, D))
